# baseline (speedup 1.0000x reference)
import functools

import jax
import jax.numpy as jnp
from jax import lax
from jax.experimental import pallas as pl
from jax.experimental.pallas import tpu as pltpu
from jax.experimental.pallas import tpu_sc as plsc

_TEMPLATE_SIZE = 100000
_TEMPLATE_FACTOR = 1000
_EMBED_DIM = 64
_BATCH = 16384
_MAX_NORM = 1.0
_SCALE = _TEMPLATE_SIZE / _TEMPLATE_FACTOR

_NC = 2
_NS = 16
_NW = _NC * _NS
_B_PER_W = _BATCH // _NW
_ROWS_PER_W = 2 * _B_PER_W
_GROUP = 128
_NGROUPS = _ROWS_PER_W // _GROUP
_LANES = 16


def _tc_prep(anno_ref, w_ref, pos_ref, iw_ref, ih_ref, table_ref):
    w2 = anno_ref[2]
    h2 = anno_ref[3]
    tw = (_SCALE * jnp.sqrt(w2 / h2)).astype(jnp.int32)
    th = (_SCALE * jnp.sqrt(h2 / w2)).astype(jnp.int32)
    iw_ref[...] = 2 * jnp.clip(tw, 0, _TEMPLATE_FACTOR - 1)
    ih_ref[...] = 2 * jnp.clip(th, 0, _TEMPLATE_FACTOR - 1) + 1

    wt = w_ref[...]
    norm = jnp.sqrt(jnp.sum(wt * wt, axis=1, keepdims=True))
    scale = jnp.where(norm > _MAX_NORM, _MAX_NORM / (norm + 1e-7),
                      jnp.ones_like(norm))
    scaled = wt * scale
    pos = pos_ref[...]
    table_ref[:, 0:_EMBED_DIM] = scaled + pos[0, 0, :][None, :]
    table_ref[:, _EMBED_DIM:2 * _EMBED_DIM] = scaled + pos[0, 1, :][None, :]


_tc_prep_call = pl.pallas_call(
    _tc_prep,
    out_shape=(
        jax.ShapeDtypeStruct((128, 128), jnp.int32),
        jax.ShapeDtypeStruct((128, 128), jnp.int32),
        jax.ShapeDtypeStruct((_TEMPLATE_FACTOR, 2 * _EMBED_DIM), jnp.float32),
    ),
)


@functools.partial(
    pl.kernel,
    mesh=plsc.VectorSubcoreMesh(core_axis_name="c", subcore_axis_name="s"),
    out_type=jax.ShapeDtypeStruct((2 * _BATCH, _EMBED_DIM), jnp.float32),
    scratch_types=[
        pltpu.VMEM((_B_PER_W,), jnp.int32),
        pltpu.VMEM((_B_PER_W,), jnp.int32),
        pltpu.VMEM((_ROWS_PER_W,), jnp.int32),
        pltpu.VMEM((_ROWS_PER_W, _EMBED_DIM), jnp.float32),
        pltpu.VMEM_SHARED((2 * _TEMPLATE_FACTOR, _EMBED_DIM), jnp.float32),
        pltpu.SemaphoreType.DMA,
        pltpu.SemaphoreType.DMA,
    ],
    compiler_params=pltpu.CompilerParams(use_tc_tiling_on_sc=False,
                                         needs_layout_passes=False),
)
def _sc_gather(table_hbm, iw_hbm, ih_hbm, out_hbm, iw_v, ih_v, il_v, rows_v,
               shared, sem, sem2):
    cid = lax.axis_index("c")
    sid = lax.axis_index("s")
    wid = sid * _NC + cid
    base = wid * _B_PER_W
    _SLAB = 2 * _TEMPLATE_FACTOR // _NS
    stage = pltpu.async_copy(table_hbm.at[pl.ds(sid * _SLAB, _SLAB)],
                             shared.at[pl.ds(sid * _SLAB, _SLAB)], sem2)
    pltpu.sync_copy(iw_hbm.at[pl.ds(base, _B_PER_W)], iw_v)
    pltpu.sync_copy(ih_hbm.at[pl.ds(base, _B_PER_W)], ih_v)
    lanes = lax.iota(jnp.int32, _LANES)
    for v in range(_B_PER_W // _LANES):
        tgt = 2 * _LANES * v + 2 * lanes
        plsc.store_scatter(il_v, [tgt], iw_v[pl.ds(v * _LANES, _LANES)])
        plsc.store_scatter(il_v, [tgt + 1], ih_v[pl.ds(v * _LANES, _LANES)])
    stage.wait()
    plsc.subcore_barrier()
    copies = []
    for g in range(_NGROUPS):
        copies.append(
            pltpu.async_copy(
                shared.at[il_v.at[pl.ds(g * _GROUP, _GROUP)]],
                rows_v.at[pl.ds(g * _GROUP, _GROUP)],
                sem,
            ))
    half = _ROWS_PER_W // 2
    for c in copies[:_NGROUPS // 2]:
        c.wait()
    out0 = pltpu.async_copy(rows_v.at[pl.ds(0, half)],
                            out_hbm.at[pl.ds(2 * base, half)], sem2)
    for c in copies[_NGROUPS // 2:]:
        c.wait()
    out0.wait()
    pltpu.sync_copy(rows_v.at[pl.ds(half, half)],
                    out_hbm.at[pl.ds(2 * base + half, half)])


_POST_CHUNK = 1024
_POST_N = _BATCH // _POST_CHUNK


def _tc_post(in_hbm, out_hbm, vin, vout, sem_in, sem_out):
    def in_copy(k):
        return pltpu.make_async_copy(
            in_hbm.at[pl.ds(k * _POST_CHUNK, _POST_CHUNK)],
            vin.at[k % 2], sem_in.at[k % 2])

    def out_copy(k, j):
        return pltpu.make_async_copy(
            vout.at[k % 2, pl.ds(j * _EMBED_DIM, _EMBED_DIM)],
            out_hbm.at[j, :, pl.ds(k * _POST_CHUNK, _POST_CHUNK)],
            sem_out.at[k % 2])

    in_copy(0).start()
    for k in range(_POST_N):
        if k + 1 < _POST_N:
            in_copy(k + 1).start()
        in_copy(k).wait()
        if k >= 2:
            out_copy(k - 2, 0).wait()
            out_copy(k - 2, 1).wait()
        x = vin[k % 2]
        vout[k % 2] = jnp.transpose(x, (1, 0))
        out_copy(k, 0).start()
        out_copy(k, 1).start()
    for k in (_POST_N - 2, _POST_N - 1):
        out_copy(k, 0).wait()
        out_copy(k, 1).wait()


_tc_post_call = pl.pallas_call(
    _tc_post,
    in_specs=[pl.BlockSpec(memory_space=pltpu.MemorySpace.HBM)],
    out_specs=pl.BlockSpec(memory_space=pltpu.MemorySpace.HBM),
    out_shape=jax.ShapeDtypeStruct((2, _EMBED_DIM, _BATCH), jnp.float32),
    scratch_shapes=[
        pltpu.VMEM((2, _POST_CHUNK, 2 * _EMBED_DIM), jnp.float32),
        pltpu.VMEM((2, 2 * _EMBED_DIM, _POST_CHUNK), jnp.float32),
        pltpu.SemaphoreType.DMA((2,)),
        pltpu.SemaphoreType.DMA((2,)),
    ],
)


def kernel(template_anno, weight, pos_embed):
    anno_t = template_anno.T.reshape(4, 128, 128)
    w1000 = weight[:_TEMPLATE_FACTOR]
    iw, ih, table128 = _tc_prep_call(anno_t, w1000, pos_embed)
    table = table128.reshape(2 * _TEMPLATE_FACTOR, _EMBED_DIM)
    out_flat = _sc_gather(table, iw.reshape(_BATCH), ih.reshape(_BATCH))
    out128 = out_flat.reshape(_BATCH, 2 * _EMBED_DIM)
    ot = _tc_post_call(out128)
    return jnp.transpose(ot, (2, 0, 1))


# speedup vs baseline: 1.1000x; 1.1000x over previous
import functools

import jax
import jax.numpy as jnp
from jax import lax
from jax.experimental import pallas as pl
from jax.experimental.pallas import tpu as pltpu
from jax.experimental.pallas import tpu_sc as plsc

_TEMPLATE_SIZE = 100000
_TEMPLATE_FACTOR = 1000
_EMBED_DIM = 64
_BATCH = 16384
_MAX_NORM = 1.0
_SCALE = _TEMPLATE_SIZE / _TEMPLATE_FACTOR

_NC = 2
_NS = 16
_NW = _NC * _NS
_B_PER_W = _BATCH // _NW
_ROWS_PER_W = 2 * _B_PER_W
_GROUP = 128
_NGROUPS = _ROWS_PER_W // _GROUP
_LANES = 16


def _tc_prep(anno_ref, w_ref, pos_ref, iw_ref, ih_ref, table_ref):
    w2 = anno_ref[2]
    h2 = anno_ref[3]
    tw = (_SCALE * jnp.sqrt(w2 / h2)).astype(jnp.int32)
    th = (_SCALE * jnp.sqrt(h2 / w2)).astype(jnp.int32)
    iw_ref[...] = 2 * jnp.clip(tw, 0, _TEMPLATE_FACTOR - 1)
    ih_ref[...] = 2 * jnp.clip(th, 0, _TEMPLATE_FACTOR - 1) + 1

    wt = w_ref[...]
    norm = jnp.sqrt(jnp.sum(wt * wt, axis=1, keepdims=True))
    scale = jnp.where(norm > _MAX_NORM, _MAX_NORM / (norm + 1e-7),
                      jnp.ones_like(norm))
    scaled = wt * scale
    pos = pos_ref[...]
    table_ref[:, 0:_EMBED_DIM] = scaled + pos[0, 0, :][None, :]
    table_ref[:, _EMBED_DIM:2 * _EMBED_DIM] = scaled + pos[0, 1, :][None, :]


_tc_prep_call = pl.pallas_call(
    _tc_prep,
    out_shape=(
        jax.ShapeDtypeStruct((128, 128), jnp.int32),
        jax.ShapeDtypeStruct((128, 128), jnp.int32),
        jax.ShapeDtypeStruct((_TEMPLATE_FACTOR, 2 * _EMBED_DIM), jnp.float32),
    ),
)


@functools.partial(
    pl.kernel,
    mesh=plsc.VectorSubcoreMesh(core_axis_name="c", subcore_axis_name="s"),
    out_type=jax.ShapeDtypeStruct((2 * _BATCH, _EMBED_DIM), jnp.float32),
    scratch_types=[
        pltpu.VMEM((_B_PER_W,), jnp.int32),
        pltpu.VMEM((_B_PER_W,), jnp.int32),
        pltpu.VMEM((_ROWS_PER_W,), jnp.int32),
        pltpu.VMEM((_ROWS_PER_W, _EMBED_DIM), jnp.float32),
        pltpu.VMEM_SHARED((2 * _TEMPLATE_FACTOR, _EMBED_DIM), jnp.float32),
        pltpu.SemaphoreType.DMA,
        pltpu.SemaphoreType.DMA,
    ],
    compiler_params=pltpu.CompilerParams(use_tc_tiling_on_sc=False,
                                         needs_layout_passes=False),
)
def _sc_gather(table_hbm, iw_hbm, ih_hbm, out_hbm, iw_v, ih_v, il_v, rows_v,
               shared, sem, sem2):
    cid = lax.axis_index("c")
    sid = lax.axis_index("s")
    wid = sid * _NC + cid
    base = wid * _B_PER_W
    _SLAB = 2 * _TEMPLATE_FACTOR // _NS
    stage = pltpu.async_copy(table_hbm.at[pl.ds(sid * _SLAB, _SLAB)],
                             shared.at[pl.ds(sid * _SLAB, _SLAB)], sem2)
    pltpu.sync_copy(iw_hbm.at[pl.ds(base, _B_PER_W)], iw_v)
    pltpu.sync_copy(ih_hbm.at[pl.ds(base, _B_PER_W)], ih_v)
    lanes = lax.iota(jnp.int32, _LANES)
    for v in range(_B_PER_W // _LANES):
        tgt = 2 * _LANES * v + 2 * lanes
        plsc.store_scatter(il_v, [tgt], iw_v[pl.ds(v * _LANES, _LANES)])
        plsc.store_scatter(il_v, [tgt + 1], ih_v[pl.ds(v * _LANES, _LANES)])
    stage.wait()
    plsc.subcore_barrier()
    copies = []
    for g in range(_NGROUPS):
        copies.append(
            pltpu.async_copy(
                shared.at[il_v.at[pl.ds(g * _GROUP, _GROUP)]],
                rows_v.at[pl.ds(g * _GROUP, _GROUP)],
                sem,
            ))
    half = _ROWS_PER_W // 2
    for c in copies[:_NGROUPS // 2]:
        c.wait()
    out0 = pltpu.async_copy(rows_v.at[pl.ds(0, half)],
                            out_hbm.at[pl.ds(2 * base, half)], sem2)
    for c in copies[_NGROUPS // 2:]:
        c.wait()
    out0.wait()
    pltpu.sync_copy(rows_v.at[pl.ds(half, half)],
                    out_hbm.at[pl.ds(2 * base + half, half)])


_POST_CHUNK = 4096
_POST_N = _BATCH // _POST_CHUNK


def _tc_post(in_hbm, out_hbm, vin, vout, sem_in, sem_out):
    def in_copy(k):
        return pltpu.make_async_copy(
            in_hbm.at[pl.ds(k * _POST_CHUNK, _POST_CHUNK)],
            vin.at[k % 2], sem_in.at[k % 2])

    def out_copy(k, j):
        return pltpu.make_async_copy(
            vout.at[k % 2, pl.ds(j * _EMBED_DIM, _EMBED_DIM)],
            out_hbm.at[j, :, pl.ds(k * _POST_CHUNK, _POST_CHUNK)],
            sem_out.at[k % 2])

    in_copy(0).start()
    for k in range(_POST_N):
        if k + 1 < _POST_N:
            in_copy(k + 1).start()
        in_copy(k).wait()
        if k >= 2:
            out_copy(k - 2, 0).wait()
            out_copy(k - 2, 1).wait()
        x = vin[k % 2]
        vout[k % 2] = jnp.transpose(x, (1, 0))
        out_copy(k, 0).start()
        out_copy(k, 1).start()
    for k in (_POST_N - 2, _POST_N - 1):
        out_copy(k, 0).wait()
        out_copy(k, 1).wait()


_tc_post_call = pl.pallas_call(
    _tc_post,
    in_specs=[pl.BlockSpec(memory_space=pltpu.MemorySpace.HBM)],
    out_specs=pl.BlockSpec(memory_space=pltpu.MemorySpace.HBM),
    out_shape=jax.ShapeDtypeStruct((2, _EMBED_DIM, _BATCH), jnp.float32),
    scratch_shapes=[
        pltpu.VMEM((2, _POST_CHUNK, 2 * _EMBED_DIM), jnp.float32),
        pltpu.VMEM((2, 2 * _EMBED_DIM, _POST_CHUNK), jnp.float32),
        pltpu.SemaphoreType.DMA((2,)),
        pltpu.SemaphoreType.DMA((2,)),
    ],
)


def kernel(template_anno, weight, pos_embed):
    anno_t = template_anno.T.reshape(4, 128, 128)
    w1000 = weight[:_TEMPLATE_FACTOR]
    iw, ih, table128 = _tc_prep_call(anno_t, w1000, pos_embed)
    table = table128.reshape(2 * _TEMPLATE_FACTOR, _EMBED_DIM)
    out_flat = _sc_gather(table, iw.reshape(_BATCH), ih.reshape(_BATCH))
    out128 = out_flat.reshape(_BATCH, 2 * _EMBED_DIM)
    ot = _tc_post_call(out128)
    return jnp.transpose(ot, (2, 0, 1))
